# Initial kernel scaffold; baseline (speedup 1.0000x reference)
import jax
import jax.numpy as jnp
from jax.experimental import pallas as pl


def kernel(edge_features, discrete_mask, emb_table, W1, b1, W2, b2):
    raise NotImplementedError("write your pallas kernel here")


# speedup vs baseline: 5.0055x; 5.0055x over previous
import functools

import jax
import jax.numpy as jnp
from jax import lax
from jax.experimental import pallas as pl
from jax.experimental.pallas import tpu as pltpu
from jax.experimental.pallas import tpu_sc as plsc

BUCKET_SIZE = 100000
EMBED_DIM = 64
EDGE_INPUT_DIM = 128
E_TOTAL = 320000
D_EDGE = 16

NC = 2
NS = 16
NW = NC * NS
EPW = E_TOTAL // NW

C = 40
ITERS = EPW // C
IDX_PER = C * D_EDGE
GATHER_W = 128
NGATH = IDX_PER // GATHER_W


def _sc_pool(feat_hbm, table_hbm, out_hbm, feat_v, idx_v, rows_v, pool_v, sem):
    wid = lax.axis_index("s") * NC + lax.axis_index("c")
    base0 = wid * EPW
    offs = (lax.iota(jnp.int32, 16) + 1) * 131

    def chunk(i, carry):
        base = base0 + i * C
        pltpu.sync_copy(feat_hbm.at[pl.ds(base, C), :], feat_v)

        def hash_row(e, c2):
            y = lax.abs(feat_v[e, :])
            n0 = y.astype(jnp.int32)
            fr = y - n0.astype(jnp.float32)
            inc = jnp.where(
                fr > jnp.float32(0.5), jnp.int32(1),
                jnp.where(fr == jnp.float32(0.5), n0 & 1, jnp.int32(0)))
            h = lax.rem(n0 + inc + offs, jnp.int32(BUCKET_SIZE))
            r = e // 8
            col = (e % 8) * 16
            idx_v[r, pl.ds(col, 16)] = h
            return c2

        lax.fori_loop(0, C, hash_row, 0)

        copies = [
            pltpu.async_copy(
                table_hbm.at[idx_v.at[j]],
                rows_v.at[pl.ds(j * GATHER_W, GATHER_W), :],
                sem,
            )
            for j in range(NGATH)
        ]
        for cp in copies:
            cp.wait()

        def acc_edge(e, c2):
            r0 = e * D_EDGE
            for s in range(EMBED_DIM // 16):
                a = rows_v[r0, pl.ds(s * 16, 16)]
                for f in range(1, D_EDGE):
                    a = a + rows_v[r0 + f, pl.ds(s * 16, 16)]
                pool_v[e, pl.ds(s * 16, 16)] = a
            return c2

        lax.fori_loop(0, C, acc_edge, 0)
        pltpu.sync_copy(pool_v, out_hbm.at[pl.ds(base, C), :])
        return carry

    lax.fori_loop(0, ITERS, chunk, 0)


_sc_pool_call = functools.partial(
    pl.kernel,
    mesh=plsc.VectorSubcoreMesh(core_axis_name="c", subcore_axis_name="s"),
    compiler_params=pltpu.CompilerParams(use_tc_tiling_on_sc=False),
    out_type=jax.ShapeDtypeStruct((E_TOTAL, EMBED_DIM), jnp.float32),
    scratch_types=[
        pltpu.VMEM((C, D_EDGE), jnp.float32),
        pltpu.VMEM((NGATH, GATHER_W), jnp.int32),
        pltpu.VMEM((IDX_PER, EMBED_DIM), jnp.float32),
        pltpu.VMEM((C, EMBED_DIM), jnp.float32),
        pltpu.SemaphoreType.DMA,
    ],
)(_sc_pool)


def _mlp_body(x_ref, w1_ref, b1_ref, w2_ref, b2_ref, o_ref):
    x = x_ref[...] * jnp.float32(1.0 / D_EDGE)
    h = jnp.dot(x, w1_ref[...], preferred_element_type=jnp.float32)
    h = jnp.maximum(h + b1_ref[...], 0.0)
    o = jnp.dot(h, w2_ref[...], preferred_element_type=jnp.float32)
    o_ref[...] = o + b2_ref[...]


BE = 3200


def _mlp(pooled, W1, b1, W2, b2):
    return pl.pallas_call(
        _mlp_body,
        grid=(E_TOTAL // BE,),
        in_specs=[
            pl.BlockSpec((BE, EMBED_DIM), lambda i: (i, 0)),
            pl.BlockSpec((EMBED_DIM, EDGE_INPUT_DIM), lambda i: (0, 0)),
            pl.BlockSpec((1, EDGE_INPUT_DIM), lambda i: (0, 0)),
            pl.BlockSpec((EDGE_INPUT_DIM, EDGE_INPUT_DIM), lambda i: (0, 0)),
            pl.BlockSpec((1, EDGE_INPUT_DIM), lambda i: (0, 0)),
        ],
        out_specs=pl.BlockSpec((BE, EDGE_INPUT_DIM), lambda i: (i, 0)),
        out_shape=jax.ShapeDtypeStruct((E_TOTAL, EDGE_INPUT_DIM), jnp.float32),
    )(pooled, W1, b1.reshape(1, -1), W2, b2.reshape(1, -1))


def kernel(edge_features, discrete_mask, emb_table, W1, b1, W2, b2):
    pooled_sum = _sc_pool_call(edge_features, emb_table)
    return _mlp(pooled_sum, W1, b1, W2, b2)
